# baseline (speedup 1.0000x reference)
import functools

import jax
import jax.numpy as jnp
from jax import lax
from jax.experimental import pallas as pl
from jax.experimental.pallas import tpu as pltpu
from jax.experimental.pallas import tpu_sc as plsc

B = 16384
HIST = 200
D = 32
CLS = 100

NC = 2
NS = 16
NW = NC * NS
BPW = B // NW
CH = 16
NCHUNK = BPW // CH
IDX_N = CH * HIST
GS0 = 104
GS1 = HIST - GS0
UNROLL = 8

_mesh = plsc.VectorSubcoreMesh(core_axis_name="c", subcore_axis_name="s")

VOCAB = 1000000
TBLK = 128
NFULL = VOCAB // TBLK
TAIL = VOCAB - NFULL * TBLK
K1_PAIRS = (NFULL // NW + 2) // 2


@functools.partial(
    pl.kernel,
    mesh=_mesh,
    out_type=jax.ShapeDtypeStruct((VOCAB * D // 2,), jnp.int32),
    compiler_params=pltpu.CompilerParams(use_tc_tiling_on_sc=True,
                                         needs_layout_passes=False),
    scratch_types=[
        pltpu.VMEM((D, TBLK), jnp.float32),
        pltpu.VMEM((D, TBLK), jnp.float32),
        pltpu.VMEM((TBLK * D // 2,), jnp.int32),
        pltpu.VMEM((TBLK * D // 2,), jnp.int32),
        pltpu.VMEM((TAIL, D), jnp.float32),
        pltpu.SemaphoreType.DMA,
        pltpu.SemaphoreType.DMA,
        pltpu.SemaphoreType.DMA,
        pltpu.SemaphoreType.DMA,
    ],
)
def _sc_relayout(embt_hbm, tail_hbm, out_hbm, in0_v, in1_v, out0_v, out1_v,
                 tail_v, isem0, isem1, osem0, osem1):
    cid = lax.axis_index("c")
    sid = lax.axis_index("s")
    wid = sid * NC + cid
    in_bufs = (in0_v, in1_v)
    out_bufs = (out0_v, out1_v)
    isems = (isem0, isem1)
    osems = (osem0, osem1)
    lane16 = lax.iota(jnp.int32, 16)

    def blk_of(k):
        return k * NW + wid

    def fire(k, b):
        @pl.when(blk_of(k) < NFULL)
        def _():
            pltpu.async_copy(
                embt_hbm.at[:, pl.ds(blk_of(k) * TBLK, TBLK)],
                in_bufs[b], isems[b])

    def consume(k, b):
        blk = blk_of(k)

        @pl.when(blk < NFULL)
        def _():
            pltpu.make_async_copy(
                embt_hbm.at[:, pl.ds(blk * TBLK, TBLK)],
                in_bufs[b], isems[b]).wait()

            @pl.when(k >= 2)
            def _():
                pltpu.make_async_copy(
                    out_bufs[b], out_hbm.at[pl.ds(0, TBLK * D // 2)],
                    osems[b]).wait()

            for c2 in range(D // 2):
                c2_vec = (c2 + lane16) & (D // 2 - 1)
                ca = c2_vec + c2_vec
                cb = ca + 1
                for ib in range(TBLK // 16):
                    t_vec = ib * 16 + lane16
                    va = plsc.load_gather(in_bufs[b], [ca, t_vec])
                    vb = plsc.load_gather(in_bufs[b], [cb, t_vec])
                    pk = plsc.pack(va, vb, format=plsc.PackFormat.INTERLEAVED)
                    w = plsc.bitcast(pk, jnp.int32)
                    plsc.store_scatter(out_bufs[b],
                                       [t_vec * (D // 2) + c2_vec], w)
            pltpu.async_copy(
                out_bufs[b],
                out_hbm.at[pl.ds(blk * (TBLK * D // 2), TBLK * D // 2)],
                osems[b])

    fire(0, 0)

    def pair_body(k2, carry):
        for b in range(2):
            k = k2 + b
            fire(k + 1, 1 - b)
            consume(k, b)
        return carry

    lax.fori_loop(0, K1_PAIRS, lambda i, cr: pair_body(i * 2, cr), 0)

    for kk in range(2 * K1_PAIRS - 4, 2 * K1_PAIRS):
        @pl.when(jnp.logical_and(blk_of(kk) < NFULL,
                                 blk_of(kk + 2) >= NFULL))
        def _():
            pltpu.make_async_copy(
                out_bufs[kk % 2], out_hbm.at[pl.ds(0, TBLK * D // 2)],
                osems[kk % 2]).wait()

    @pl.when(wid == 0)
    def _():
        pltpu.sync_copy(tail_hbm, tail_v)
        for t in range(TAIL):
            fa = plsc.load_gather(tail_v, [jnp.full((16,), t, jnp.int32),
                                           lane16 + lane16])
            fb = plsc.load_gather(tail_v, [jnp.full((16,), t, jnp.int32),
                                           lane16 + lane16 + 1])
            pk = plsc.pack(fa, fb, format=plsc.PackFormat.INTERLEAVED)
            w = plsc.bitcast(pk, jnp.int32)
            out0_v[pl.ds(t * (D // 2), 16)] = w
        pltpu.sync_copy(out0_v.at[pl.ds(0, TAIL * D // 2)],
                        out_hbm.at[pl.ds(NFULL * TBLK * D // 2,
                                         TAIL * D // 2)])


@functools.partial(
    pl.kernel,
    mesh=_mesh,
    out_type=jax.ShapeDtypeStruct((B, D), jnp.float32),
    compiler_params=pltpu.CompilerParams(use_tc_tiling_on_sc=False,
                                         needs_layout_passes=False),
    scratch_types=[
        pltpu.VMEM((2, CH, HIST), jnp.float32),
        pltpu.VMEM((2, CH, HIST), jnp.int32),
        pltpu.VMEM((IDX_N, D // 2), jnp.int32),
        pltpu.VMEM((IDX_N, D // 2), jnp.int32),
        pltpu.VMEM((CH, D), jnp.float32),
        pltpu.SemaphoreType.DMA,
        pltpu.SemaphoreType.DMA,
    ],
)
def _sc_bag_sum(xf_hbm, emb_hbm, out_hbm, idxf_v, idxi_v,
                rows0_v, rows1_v, sum_v, sem0, sem1):
    cid = lax.axis_index("c")
    sid = lax.axis_index("s")
    wid = sid * NC + cid
    base = wid * BPW
    rows_bufs = (rows0_v, rows1_v)
    sems = (sem0, sem1)

    def gather_list(b):
        out = []
        for s in range(CH):
            out.append((idxi_v.at[b, s, pl.ds(0, GS0)],
                        rows_bufs[b].at[pl.ds(s * HIST, GS0)]))
            out.append((idxi_v.at[b, s, pl.ds(GS0, GS1)],
                        rows_bufs[b].at[pl.ds(s * HIST + GS0, GS1)]))
        return out

    def fire(c, b):
        off = base + c * CH
        pltpu.sync_copy(xf_hbm.at[pl.ds(off, CH), :], idxf_v.at[b])
        for s in range(CH):
            for k in range(HIST // 16 + 1):
                col = min(k * 16, HIST - 16)
                v = idxf_v[b, s, pl.ds(col, 16)]
                idxi_v[b, s, pl.ds(col, 16)] = plsc.bitcast(v, jnp.int32)
        for isl, rsl in gather_list(b):
            pltpu.async_copy(emb_hbm.at[isl], rsl, sems[b])

    def drain(b):
        for isl, rsl in gather_list(b):
            pltpu.make_async_copy(emb_hbm.at[isl], rsl, sems[b]).wait()

    def consume(c, b):
        rows_v = rows_bufs[b]
        for s in range(CH):
            def row_body(j, accs):
                accs = list(accs)
                r = s * HIST + j * UNROLL
                for u in range(UNROLL):
                    w = rows_v[r + u, pl.ds(0, 16)]
                    ev, od = plsc.unpack(
                        plsc.bitcast(w, jnp.bfloat16),
                        format=plsc.PackFormat.INTERLEAVED)
                    q = u % 4
                    accs[2 * q] = accs[2 * q] + ev
                    accs[2 * q + 1] = accs[2 * q + 1] + od
                return tuple(accs)

            zero = jnp.zeros((16,), jnp.float32)
            accs = lax.fori_loop(0, HIST // UNROLL, row_body, (zero,) * 8)
            sum_v[s, pl.ds(0, 16)] = (accs[0] + accs[2]) + (accs[4] + accs[6])
            sum_v[s, pl.ds(16, 16)] = (accs[1] + accs[3]) + (accs[5] + accs[7])
        off = base + c * CH
        pltpu.sync_copy(sum_v, out_hbm.at[pl.ds(off, CH)])

    fire(0, 0)

    def pair_body(c2, carry):
        for b in range(2):
            c = c2 + b

            @pl.when(c + 1 < NCHUNK)
            def _():
                fire(c + 1, 1 - b)

            drain(b)
            consume(c, b)
        return carry

    lax.fori_loop(0, NCHUNK // 2, lambda i, cr: pair_body(i * 2, cr), 0)


_TC_BLK = 2048


def _tc_head_body(x_ref, sum_ref, w_ref, b_ref, o_ref):
    cnt = jnp.sum((x_ref[...] != 0).astype(jnp.float32), axis=1,
                  keepdims=True)
    mean = sum_ref[...] * (1.0 / jnp.maximum(cnt, 1.0))
    o_ref[...] = (
        jnp.dot(mean, w_ref[...], preferred_element_type=jnp.float32)
        + b_ref[...])


_tc_head = pl.pallas_call(
    _tc_head_body,
    grid=(B // _TC_BLK,),
    in_specs=[
        pl.BlockSpec((_TC_BLK, HIST), lambda i: (i, 0)),
        pl.BlockSpec((_TC_BLK, D), lambda i: (i, 0)),
        pl.BlockSpec((D, CLS), lambda i: (0, 0)),
        pl.BlockSpec((1, CLS), lambda i: (0, 0)),
    ],
    out_specs=pl.BlockSpec((_TC_BLK, CLS), lambda i: (i, 0)),
    out_shape=jax.ShapeDtypeStruct((B, CLS), jnp.float32),
)


def kernel(x, emb, W, b):
    x = x.astype(jnp.int32)
    xf = jax.lax.bitcast_convert_type(x, jnp.float32)
    emb_lin = _sc_relayout(emb.T, emb[NFULL * TBLK:, :]).reshape(VOCAB,
                                                                 D // 2)
    summed = _sc_bag_sum(xf, emb_lin)
    w_eo = jnp.concatenate([W[0::2, :], W[1::2, :]], axis=0)
    return _tc_head(x, summed, w_eo, b.reshape(1, CLS))


# speedup vs baseline: 1.4314x; 1.4314x over previous
import functools

import jax
import jax.numpy as jnp
from jax import lax
from jax.experimental import pallas as pl
from jax.experimental.pallas import tpu as pltpu
from jax.experimental.pallas import tpu_sc as plsc

B = 16384
HIST = 200
D = 32
CLS = 100

NC = 2
NS = 16
NW = NC * NS
BPW = B // NW
CH = 16
NCHUNK = BPW // CH
IDX_N = CH * HIST
GS0 = 104
GS1 = HIST - GS0
UNROLL = 8

_mesh = plsc.VectorSubcoreMesh(core_axis_name="c", subcore_axis_name="s")

VOCAB = 1000000
TBLK = 128
NFULL = VOCAB // TBLK
TAIL = VOCAB - NFULL * TBLK
K1_PAIRS = (NFULL // NW + 2) // 2


@functools.partial(
    pl.kernel,
    mesh=_mesh,
    out_type=jax.ShapeDtypeStruct((VOCAB * D // 2,), jnp.int32),
    compiler_params=pltpu.CompilerParams(use_tc_tiling_on_sc=True,
                                         needs_layout_passes=False),
    scratch_types=[
        pltpu.VMEM((D, TBLK), jnp.float32),
        pltpu.VMEM((D, TBLK), jnp.float32),
        pltpu.VMEM((TBLK * D // 2,), jnp.int32),
        pltpu.VMEM((TBLK * D // 2,), jnp.int32),
        pltpu.VMEM((TAIL, D), jnp.float32),
        pltpu.SemaphoreType.DMA,
        pltpu.SemaphoreType.DMA,
        pltpu.SemaphoreType.DMA,
        pltpu.SemaphoreType.DMA,
    ],
)
def _sc_relayout(embt_hbm, tail_hbm, out_hbm, in0_v, in1_v, out0_v, out1_v,
                 tail_v, isem0, isem1, osem0, osem1):
    cid = lax.axis_index("c")
    sid = lax.axis_index("s")
    wid = sid * NC + cid
    in_bufs = (in0_v, in1_v)
    out_bufs = (out0_v, out1_v)
    isems = (isem0, isem1)
    osems = (osem0, osem1)
    lane16 = lax.iota(jnp.int32, 16)

    def blk_of(k):
        return k * NW + wid

    def fire(k, b):
        @pl.when(blk_of(k) < NFULL)
        def _():
            pltpu.async_copy(
                embt_hbm.at[:, pl.ds(blk_of(k) * TBLK, TBLK)],
                in_bufs[b], isems[b])

    def consume(k, b):
        blk = blk_of(k)

        @pl.when(blk < NFULL)
        def _():
            pltpu.make_async_copy(
                embt_hbm.at[:, pl.ds(blk * TBLK, TBLK)],
                in_bufs[b], isems[b]).wait()

            @pl.when(k >= 2)
            def _():
                pltpu.make_async_copy(
                    out_bufs[b], out_hbm.at[pl.ds(0, TBLK * D // 2)],
                    osems[b]).wait()

            def tr_body(ib, carry):
                t_vec = ib * 16 + lane16
                t_h = t_vec * (D // 2)
                for c2 in range(D // 2):
                    c2_vec = (c2 + lane16) & (D // 2 - 1)
                    ca = c2_vec + c2_vec
                    va = plsc.load_gather(in_bufs[b], [ca, t_vec])
                    vb = plsc.load_gather(in_bufs[b], [ca + 1, t_vec])
                    pk = plsc.pack(va, vb, format=plsc.PackFormat.INTERLEAVED)
                    w = plsc.bitcast(pk, jnp.int32)
                    plsc.store_scatter(out_bufs[b], [t_h + c2_vec], w)
                return carry

            lax.fori_loop(0, TBLK // 16, tr_body, 0)
            pltpu.async_copy(
                out_bufs[b],
                out_hbm.at[pl.ds(blk * (TBLK * D // 2), TBLK * D // 2)],
                osems[b])

    fire(0, 0)

    def pair_body(k2, carry):
        for b in range(2):
            k = k2 + b
            fire(k + 1, 1 - b)
            consume(k, b)
        return carry

    lax.fori_loop(0, K1_PAIRS, lambda i, cr: pair_body(i * 2, cr), 0)

    for kk in range(2 * K1_PAIRS - 4, 2 * K1_PAIRS):
        @pl.when(jnp.logical_and(blk_of(kk) < NFULL,
                                 blk_of(kk + 2) >= NFULL))
        def _():
            pltpu.make_async_copy(
                out_bufs[kk % 2], out_hbm.at[pl.ds(0, TBLK * D // 2)],
                osems[kk % 2]).wait()

    @pl.when(wid == 0)
    def _():
        pltpu.sync_copy(tail_hbm, tail_v)
        for t in range(TAIL):
            fa = plsc.load_gather(tail_v, [jnp.full((16,), t, jnp.int32),
                                           lane16 + lane16])
            fb = plsc.load_gather(tail_v, [jnp.full((16,), t, jnp.int32),
                                           lane16 + lane16 + 1])
            pk = plsc.pack(fa, fb, format=plsc.PackFormat.INTERLEAVED)
            w = plsc.bitcast(pk, jnp.int32)
            out0_v[pl.ds(t * (D // 2), 16)] = w
        pltpu.sync_copy(out0_v.at[pl.ds(0, TAIL * D // 2)],
                        out_hbm.at[pl.ds(NFULL * TBLK * D // 2,
                                         TAIL * D // 2)])


@functools.partial(
    pl.kernel,
    mesh=_mesh,
    out_type=jax.ShapeDtypeStruct((B, D), jnp.float32),
    compiler_params=pltpu.CompilerParams(use_tc_tiling_on_sc=False,
                                         needs_layout_passes=False),
    scratch_types=[
        pltpu.VMEM((2, CH, HIST), jnp.float32),
        pltpu.VMEM((2, CH, HIST), jnp.int32),
        pltpu.VMEM((IDX_N, D // 2), jnp.int32),
        pltpu.VMEM((IDX_N, D // 2), jnp.int32),
        pltpu.VMEM((CH, D), jnp.float32),
        pltpu.SemaphoreType.DMA,
        pltpu.SemaphoreType.DMA,
    ],
)
def _sc_bag_sum(xf_hbm, emb_hbm, out_hbm, idxf_v, idxi_v,
                rows0_v, rows1_v, sum_v, sem0, sem1):
    cid = lax.axis_index("c")
    sid = lax.axis_index("s")
    wid = sid * NC + cid
    base = wid * BPW
    rows_bufs = (rows0_v, rows1_v)
    sems = (sem0, sem1)

    def gather_list(b):
        out = []
        for s in range(CH):
            out.append((idxi_v.at[b, s, pl.ds(0, GS0)],
                        rows_bufs[b].at[pl.ds(s * HIST, GS0)]))
            out.append((idxi_v.at[b, s, pl.ds(GS0, GS1)],
                        rows_bufs[b].at[pl.ds(s * HIST + GS0, GS1)]))
        return out

    def fire(c, b):
        off = base + c * CH
        pltpu.sync_copy(xf_hbm.at[pl.ds(off, CH), :], idxf_v.at[b])
        for s in range(CH):
            for k in range(HIST // 16 + 1):
                col = min(k * 16, HIST - 16)
                v = idxf_v[b, s, pl.ds(col, 16)]
                idxi_v[b, s, pl.ds(col, 16)] = plsc.bitcast(v, jnp.int32)
        for isl, rsl in gather_list(b):
            pltpu.async_copy(emb_hbm.at[isl], rsl, sems[b])

    def drain(b):
        for isl, rsl in gather_list(b):
            pltpu.make_async_copy(emb_hbm.at[isl], rsl, sems[b]).wait()

    def consume(c, b):
        rows_v = rows_bufs[b]
        for s in range(CH):
            def row_body(j, accs):
                accs = list(accs)
                r = s * HIST + j * UNROLL
                for u in range(UNROLL):
                    w = rows_v[r + u, pl.ds(0, 16)]
                    ev, od = plsc.unpack(
                        plsc.bitcast(w, jnp.bfloat16),
                        format=plsc.PackFormat.INTERLEAVED)
                    q = u % 4
                    accs[2 * q] = accs[2 * q] + ev
                    accs[2 * q + 1] = accs[2 * q + 1] + od
                return tuple(accs)

            zero = jnp.zeros((16,), jnp.float32)
            accs = lax.fori_loop(0, HIST // UNROLL, row_body, (zero,) * 8)
            sum_v[s, pl.ds(0, 16)] = (accs[0] + accs[2]) + (accs[4] + accs[6])
            sum_v[s, pl.ds(16, 16)] = (accs[1] + accs[3]) + (accs[5] + accs[7])
        off = base + c * CH
        pltpu.sync_copy(sum_v, out_hbm.at[pl.ds(off, CH)])

    fire(0, 0)

    def pair_body(c2, carry):
        for b in range(2):
            c = c2 + b

            @pl.when(c + 1 < NCHUNK)
            def _():
                fire(c + 1, 1 - b)

            drain(b)
            consume(c, b)
        return carry

    lax.fori_loop(0, NCHUNK // 2, lambda i, cr: pair_body(i * 2, cr), 0)


_TC_BLK = 2048


def _tc_head_body(x_ref, sum_ref, w_ref, b_ref, o_ref):
    cnt = jnp.sum((x_ref[...] != 0).astype(jnp.float32), axis=1,
                  keepdims=True)
    mean = sum_ref[...] * (1.0 / jnp.maximum(cnt, 1.0))
    o_ref[...] = (
        jnp.dot(mean, w_ref[...], preferred_element_type=jnp.float32)
        + b_ref[...])


_tc_head = pl.pallas_call(
    _tc_head_body,
    grid=(B // _TC_BLK,),
    in_specs=[
        pl.BlockSpec((_TC_BLK, HIST), lambda i: (i, 0)),
        pl.BlockSpec((_TC_BLK, D), lambda i: (i, 0)),
        pl.BlockSpec((D, CLS), lambda i: (0, 0)),
        pl.BlockSpec((1, CLS), lambda i: (0, 0)),
    ],
    out_specs=pl.BlockSpec((_TC_BLK, CLS), lambda i: (i, 0)),
    out_shape=jax.ShapeDtypeStruct((B, CLS), jnp.float32),
)


def kernel(x, emb, W, b):
    x = x.astype(jnp.int32)
    xf = jax.lax.bitcast_convert_type(x, jnp.float32)
    emb_lin = _sc_relayout(emb.T, emb[NFULL * TBLK:, :]).reshape(VOCAB,
                                                                 D // 2)
    summed = _sc_bag_sum(xf, emb_lin)
    w_eo = jnp.concatenate([W[0::2, :], W[1::2, :]], axis=0)
    return _tc_head(x, summed, w_eo, b.reshape(1, CLS))


# speedup vs baseline: 1.4631x; 1.0221x over previous
import functools

import jax
import jax.numpy as jnp
from jax import lax
from jax.experimental import pallas as pl
from jax.experimental.pallas import tpu as pltpu
from jax.experimental.pallas import tpu_sc as plsc

B = 16384
HIST = 200
D = 32
CLS = 100

NC = 2
NS = 16
NW = NC * NS
BPW = B // NW
CH = 16
NCHUNK = BPW // CH
IDX_N = CH * HIST
GS0 = 104
GS1 = HIST - GS0
UNROLL = 8

_mesh = plsc.VectorSubcoreMesh(core_axis_name="c", subcore_axis_name="s")

VOCAB = 1000000
TBLK = 128
NFULL = VOCAB // TBLK
TAIL = VOCAB - NFULL * TBLK
K1_PAIRS = (NFULL // NW + 2) // 2


@functools.partial(
    pl.kernel,
    mesh=_mesh,
    out_type=jax.ShapeDtypeStruct((VOCAB * D // 2,), jnp.int32),
    compiler_params=pltpu.CompilerParams(use_tc_tiling_on_sc=True,
                                         needs_layout_passes=False),
    scratch_types=[
        pltpu.VMEM((D, TBLK), jnp.float32),
        pltpu.VMEM((D, TBLK), jnp.float32),
        pltpu.VMEM((TBLK * D // 2,), jnp.int32),
        pltpu.VMEM((TBLK * D // 2,), jnp.int32),
        pltpu.VMEM((TAIL, D), jnp.float32),
        pltpu.SemaphoreType.DMA,
        pltpu.SemaphoreType.DMA,
        pltpu.SemaphoreType.DMA,
        pltpu.SemaphoreType.DMA,
    ],
)
def _sc_relayout(embt_hbm, tail_hbm, out_hbm, in0_v, in1_v, out0_v, out1_v,
                 tail_v, isem0, isem1, osem0, osem1):
    cid = lax.axis_index("c")
    sid = lax.axis_index("s")
    wid = sid * NC + cid
    in_bufs = (in0_v, in1_v)
    out_bufs = (out0_v, out1_v)
    isems = (isem0, isem1)
    osems = (osem0, osem1)
    lane16 = lax.iota(jnp.int32, 16)

    def blk_of(k):
        return k * NW + wid

    def fire(k, b):
        @pl.when(blk_of(k) < NFULL)
        def _():
            pltpu.async_copy(
                embt_hbm.at[:, pl.ds(blk_of(k) * TBLK, TBLK)],
                in_bufs[b], isems[b])

    def consume(k, b):
        blk = blk_of(k)

        @pl.when(blk < NFULL)
        def _():
            pltpu.make_async_copy(
                embt_hbm.at[:, pl.ds(blk * TBLK, TBLK)],
                in_bufs[b], isems[b]).wait()

            @pl.when(k >= 2)
            def _():
                pltpu.make_async_copy(
                    out_bufs[b], out_hbm.at[pl.ds(0, TBLK * D // 2)],
                    osems[b]).wait()

            def tr_body(ib, carry):
                t_vec = ib * 16 + lane16
                t_h = t_vec * (D // 2)
                for c2 in range(D // 2):
                    c2_vec = (c2 + lane16) & (D // 2 - 1)
                    ca = c2_vec + c2_vec
                    va = plsc.load_gather(in_bufs[b], [ca, t_vec])
                    vb = plsc.load_gather(in_bufs[b], [ca + 1, t_vec])
                    pk = plsc.pack(va, vb, format=plsc.PackFormat.INTERLEAVED)
                    w = plsc.bitcast(pk, jnp.int32)
                    plsc.store_scatter(out_bufs[b], [t_h + c2_vec], w)
                return carry

            lax.fori_loop(0, TBLK // 16, tr_body, 0)
            pltpu.async_copy(
                out_bufs[b],
                out_hbm.at[pl.ds(blk * (TBLK * D // 2), TBLK * D // 2)],
                osems[b])

    fire(0, 0)

    def pair_body(k2, carry):
        for b in range(2):
            k = k2 + b
            fire(k + 1, 1 - b)
            consume(k, b)
        return carry

    lax.fori_loop(0, K1_PAIRS, lambda i, cr: pair_body(i * 2, cr), 0)

    for kk in range(2 * K1_PAIRS - 4, 2 * K1_PAIRS):
        @pl.when(jnp.logical_and(blk_of(kk) < NFULL,
                                 blk_of(kk + 2) >= NFULL))
        def _():
            pltpu.make_async_copy(
                out_bufs[kk % 2], out_hbm.at[pl.ds(0, TBLK * D // 2)],
                osems[kk % 2]).wait()

    @pl.when(wid == 0)
    def _():
        pltpu.sync_copy(tail_hbm, tail_v)
        for t in range(TAIL):
            fa = plsc.load_gather(tail_v, [jnp.full((16,), t, jnp.int32),
                                           lane16 + lane16])
            fb = plsc.load_gather(tail_v, [jnp.full((16,), t, jnp.int32),
                                           lane16 + lane16 + 1])
            pk = plsc.pack(fa, fb, format=plsc.PackFormat.INTERLEAVED)
            w = plsc.bitcast(pk, jnp.int32)
            out0_v[pl.ds(t * (D // 2), 16)] = w
        pltpu.sync_copy(out0_v.at[pl.ds(0, TAIL * D // 2)],
                        out_hbm.at[pl.ds(NFULL * TBLK * D // 2,
                                         TAIL * D // 2)])


@functools.partial(
    pl.kernel,
    mesh=_mesh,
    out_type=jax.ShapeDtypeStruct((B, D), jnp.float32),
    compiler_params=pltpu.CompilerParams(use_tc_tiling_on_sc=False,
                                         needs_layout_passes=False),
    scratch_types=[
        pltpu.VMEM((2, CH, HIST), jnp.float32),
        pltpu.VMEM((2, CH, HIST), jnp.int32),
        pltpu.VMEM((IDX_N, D // 2), jnp.int32),
        pltpu.VMEM((IDX_N, D // 2), jnp.int32),
        pltpu.VMEM((CH, D), jnp.float32),
        pltpu.SemaphoreType.DMA,
        pltpu.SemaphoreType.DMA,
        pltpu.SemaphoreType.DMA,
        pltpu.SemaphoreType.DMA,
    ],
)
def _sc_bag_sum(xf_hbm, emb_hbm, out_hbm, idxf_v, idxi_v,
                rows0_v, rows1_v, sum_v, sem0, sem1, xsem0, xsem1):
    cid = lax.axis_index("c")
    sid = lax.axis_index("s")
    wid = sid * NC + cid
    base = wid * BPW
    rows_bufs = (rows0_v, rows1_v)
    sems = (sem0, sem1)
    xsems = (xsem0, xsem1)

    def gather_list(b):
        out = []
        for s in range(CH):
            out.append((idxi_v.at[b, s, pl.ds(0, GS0)],
                        rows_bufs[b].at[pl.ds(s * HIST, GS0)]))
            out.append((idxi_v.at[b, s, pl.ds(GS0, GS1)],
                        rows_bufs[b].at[pl.ds(s * HIST + GS0, GS1)]))
        return out

    def stage(c, b):
        @pl.when(c < NCHUNK)
        def _():
            off = base + c * CH
            pltpu.async_copy(xf_hbm.at[pl.ds(off, CH), :], idxf_v.at[b],
                             xsems[b])

    def convert(b):
        for s in range(CH):
            for k in range(HIST // 16 + 1):
                col = min(k * 16, HIST - 16)
                v = idxf_v[b, s, pl.ds(col, 16)]
                idxi_v[b, s, pl.ds(col, 16)] = plsc.bitcast(v, jnp.int32)

    def fire(c, b):
        @pl.when(c < NCHUNK)
        def _():
            pltpu.make_async_copy(xf_hbm.at[pl.ds(base, CH), :],
                                  idxf_v.at[b], xsems[b]).wait()
            convert(b)
            for isl, rsl in gather_list(b):
                pltpu.async_copy(emb_hbm.at[isl], rsl, sems[b])

    def drain(b):
        for isl, rsl in gather_list(b):
            pltpu.make_async_copy(emb_hbm.at[isl], rsl, sems[b]).wait()

    def consume(c, b):
        rows_v = rows_bufs[b]
        for s in range(CH):
            def row_body(j, accs):
                accs = list(accs)
                r = s * HIST + j * UNROLL
                for u in range(UNROLL):
                    w = rows_v[r + u, pl.ds(0, 16)]
                    ev, od = plsc.unpack(
                        plsc.bitcast(w, jnp.bfloat16),
                        format=plsc.PackFormat.INTERLEAVED)
                    q = u % 4
                    accs[2 * q] = accs[2 * q] + ev
                    accs[2 * q + 1] = accs[2 * q + 1] + od
                return tuple(accs)

            zero = jnp.zeros((16,), jnp.float32)
            accs = lax.fori_loop(0, HIST // UNROLL, row_body, (zero,) * 8)
            sum_v[s, pl.ds(0, 16)] = (accs[0] + accs[2]) + (accs[4] + accs[6])
            sum_v[s, pl.ds(16, 16)] = (accs[1] + accs[3]) + (accs[5] + accs[7])
        off = base + c * CH
        pltpu.sync_copy(sum_v, out_hbm.at[pl.ds(off, CH)])

    pltpu.sync_copy(xf_hbm.at[pl.ds(base, CH), :], idxf_v.at[0])
    convert(0)
    for _isl, _rsl in gather_list(0):
        pltpu.async_copy(emb_hbm.at[_isl], _rsl, sems[0])
    stage(1, 1)

    def pair_body(c2, carry):
        for b in range(2):
            c = c2 + b
            fire(c + 1, 1 - b)
            stage(c + 2, b)
            drain(b)
            consume(c, b)
        return carry

    lax.fori_loop(0, NCHUNK // 2, lambda i, cr: pair_body(i * 2, cr), 0)


_TC_BLK = 2048


def _tc_head_body(x_ref, sum_ref, w_ref, b_ref, o_ref):
    cnt = jnp.sum((x_ref[...] != 0).astype(jnp.float32), axis=1,
                  keepdims=True)
    mean = sum_ref[...] * (1.0 / jnp.maximum(cnt, 1.0))
    o_ref[...] = (
        jnp.dot(mean, w_ref[...], preferred_element_type=jnp.float32)
        + b_ref[...])


_tc_head = pl.pallas_call(
    _tc_head_body,
    grid=(B // _TC_BLK,),
    in_specs=[
        pl.BlockSpec((_TC_BLK, HIST), lambda i: (i, 0)),
        pl.BlockSpec((_TC_BLK, D), lambda i: (i, 0)),
        pl.BlockSpec((D, CLS), lambda i: (0, 0)),
        pl.BlockSpec((1, CLS), lambda i: (0, 0)),
    ],
    out_specs=pl.BlockSpec((_TC_BLK, CLS), lambda i: (i, 0)),
    out_shape=jax.ShapeDtypeStruct((B, CLS), jnp.float32),
)


def kernel(x, emb, W, b):
    x = x.astype(jnp.int32)
    xf = jax.lax.bitcast_convert_type(x, jnp.float32)
    emb_lin = _sc_relayout(emb.T, emb[NFULL * TBLK:, :]).reshape(VOCAB,
                                                                 D // 2)
    summed = _sc_bag_sum(xf, emb_lin)
    w_eo = jnp.concatenate([W[0::2, :], W[1::2, :]], axis=0)
    return _tc_head(x, summed, w_eo, b.reshape(1, CLS))


# speedup vs baseline: 1.5756x; 1.0769x over previous
import functools

import jax
import jax.numpy as jnp
from jax import lax
from jax.experimental import pallas as pl
from jax.experimental.pallas import tpu as pltpu
from jax.experimental.pallas import tpu_sc as plsc

B = 16384
HIST = 200
D = 32
CLS = 100

NC = 2
NS = 16
NW = NC * NS
BPW = B // NW
CH = 16
NCHUNK = BPW // CH
IDX_N = CH * HIST
GS0 = 104
GS1 = HIST - GS0
UNROLL = 8

_mesh = plsc.VectorSubcoreMesh(core_axis_name="c", subcore_axis_name="s")

VOCAB = 1000000
TBLK = 256
NFULL = VOCAB // TBLK
TAIL = VOCAB - NFULL * TBLK
K1_PAIRS = (NFULL // NW + 2) // 2


@functools.partial(
    pl.kernel,
    mesh=_mesh,
    out_type=jax.ShapeDtypeStruct((VOCAB * D // 2,), jnp.int32),
    compiler_params=pltpu.CompilerParams(use_tc_tiling_on_sc=True,
                                         needs_layout_passes=False),
    scratch_types=[
        pltpu.VMEM((D, TBLK), jnp.float32),
        pltpu.VMEM((D, TBLK), jnp.float32),
        pltpu.VMEM((TBLK * D // 2,), jnp.int32),
        pltpu.VMEM((TBLK * D // 2,), jnp.int32),
        pltpu.VMEM((TAIL, D), jnp.float32),
        pltpu.SemaphoreType.DMA,
        pltpu.SemaphoreType.DMA,
        pltpu.SemaphoreType.DMA,
        pltpu.SemaphoreType.DMA,
    ],
)
def _sc_relayout(embt_hbm, tail_hbm, out_hbm, in0_v, in1_v, out0_v, out1_v,
                 tail_v, isem0, isem1, osem0, osem1):
    cid = lax.axis_index("c")
    sid = lax.axis_index("s")
    wid = sid * NC + cid
    in_bufs = (in0_v, in1_v)
    out_bufs = (out0_v, out1_v)
    isems = (isem0, isem1)
    osems = (osem0, osem1)
    lane16 = lax.iota(jnp.int32, 16)

    def blk_of(k):
        return k * NW + wid

    def fire(k, b):
        @pl.when(blk_of(k) < NFULL)
        def _():
            pltpu.async_copy(
                embt_hbm.at[:, pl.ds(blk_of(k) * TBLK, TBLK)],
                in_bufs[b], isems[b])

    def consume(k, b):
        blk = blk_of(k)

        @pl.when(blk < NFULL)
        def _():
            pltpu.make_async_copy(
                embt_hbm.at[:, pl.ds(blk * TBLK, TBLK)],
                in_bufs[b], isems[b]).wait()

            @pl.when(k >= 2)
            def _():
                pltpu.make_async_copy(
                    out_bufs[b], out_hbm.at[pl.ds(0, TBLK * D // 2)],
                    osems[b]).wait()

            def tr_body(ib, carry):
                t_vec = ib * 16 + lane16
                t_h = t_vec * (D // 2)
                for c2 in range(D // 2):
                    c2_vec = (c2 + lane16) & (D // 2 - 1)
                    ca = c2_vec + c2_vec
                    va = plsc.load_gather(in_bufs[b], [ca, t_vec])
                    vb = plsc.load_gather(in_bufs[b], [ca + 1, t_vec])
                    pk = plsc.pack(va, vb, format=plsc.PackFormat.INTERLEAVED)
                    w = plsc.bitcast(pk, jnp.int32)
                    plsc.store_scatter(out_bufs[b], [t_h + c2_vec], w)
                return carry

            lax.fori_loop(0, TBLK // 16, tr_body, 0)
            pltpu.async_copy(
                out_bufs[b],
                out_hbm.at[pl.ds(blk * (TBLK * D // 2), TBLK * D // 2)],
                osems[b])

    fire(0, 0)

    def pair_body(k2, carry):
        for b in range(2):
            k = k2 + b
            fire(k + 1, 1 - b)
            consume(k, b)
        return carry

    lax.fori_loop(0, K1_PAIRS, lambda i, cr: pair_body(i * 2, cr), 0)

    for kk in range(2 * K1_PAIRS - 4, 2 * K1_PAIRS):
        @pl.when(jnp.logical_and(blk_of(kk) < NFULL,
                                 blk_of(kk + 2) >= NFULL))
        def _():
            pltpu.make_async_copy(
                out_bufs[kk % 2], out_hbm.at[pl.ds(0, TBLK * D // 2)],
                osems[kk % 2]).wait()

    @pl.when(wid == 0)
    def _():
        pltpu.sync_copy(tail_hbm, tail_v)
        for t in range(TAIL):
            fa = plsc.load_gather(tail_v, [jnp.full((16,), t, jnp.int32),
                                           lane16 + lane16])
            fb = plsc.load_gather(tail_v, [jnp.full((16,), t, jnp.int32),
                                           lane16 + lane16 + 1])
            pk = plsc.pack(fa, fb, format=plsc.PackFormat.INTERLEAVED)
            w = plsc.bitcast(pk, jnp.int32)
            out0_v[pl.ds(t * (D // 2), 16)] = w
        pltpu.sync_copy(out0_v.at[pl.ds(0, TAIL * D // 2)],
                        out_hbm.at[pl.ds(NFULL * TBLK * D // 2,
                                         TAIL * D // 2)])


@functools.partial(
    pl.kernel,
    mesh=_mesh,
    out_type=jax.ShapeDtypeStruct((B, D), jnp.float32),
    compiler_params=pltpu.CompilerParams(use_tc_tiling_on_sc=False,
                                         needs_layout_passes=False),
    scratch_types=[
        pltpu.VMEM((2, CH, HIST), jnp.float32),
        pltpu.VMEM((2, CH, HIST), jnp.int32),
        pltpu.VMEM((IDX_N, D // 2), jnp.int32),
        pltpu.VMEM((IDX_N, D // 2), jnp.int32),
        pltpu.VMEM((CH, D), jnp.float32),
        pltpu.SemaphoreType.DMA,
        pltpu.SemaphoreType.DMA,
        pltpu.SemaphoreType.DMA,
        pltpu.SemaphoreType.DMA,
    ],
)
def _sc_bag_sum(xf_hbm, emb_hbm, out_hbm, idxf_v, idxi_v,
                rows0_v, rows1_v, sum_v, sem0, sem1, xsem0, xsem1):
    cid = lax.axis_index("c")
    sid = lax.axis_index("s")
    wid = sid * NC + cid
    base = wid * BPW
    rows_bufs = (rows0_v, rows1_v)
    sems = (sem0, sem1)
    xsems = (xsem0, xsem1)

    def gather_list(b):
        out = []
        for s in range(CH):
            out.append((idxi_v.at[b, s, pl.ds(0, GS0)],
                        rows_bufs[b].at[pl.ds(s * HIST, GS0)]))
            out.append((idxi_v.at[b, s, pl.ds(GS0, GS1)],
                        rows_bufs[b].at[pl.ds(s * HIST + GS0, GS1)]))
        return out

    def stage(c, b):
        @pl.when(c < NCHUNK)
        def _():
            off = base + c * CH
            pltpu.async_copy(xf_hbm.at[pl.ds(off, CH), :], idxf_v.at[b],
                             xsems[b])

    def convert(b):
        for s in range(CH):
            for k in range(HIST // 16 + 1):
                col = min(k * 16, HIST - 16)
                v = idxf_v[b, s, pl.ds(col, 16)]
                idxi_v[b, s, pl.ds(col, 16)] = plsc.bitcast(v, jnp.int32)

    def fire(c, b):
        @pl.when(c < NCHUNK)
        def _():
            pltpu.make_async_copy(xf_hbm.at[pl.ds(base, CH), :],
                                  idxf_v.at[b], xsems[b]).wait()
            convert(b)
            for isl, rsl in gather_list(b):
                pltpu.async_copy(emb_hbm.at[isl], rsl, sems[b])

    def drain(b):
        for isl, rsl in gather_list(b):
            pltpu.make_async_copy(emb_hbm.at[isl], rsl, sems[b]).wait()

    def consume(c, b):
        rows_v = rows_bufs[b]
        for s in range(CH):
            def row_body(j, accs):
                accs = list(accs)
                r = s * HIST + j * UNROLL
                for u in range(UNROLL):
                    w = rows_v[r + u, pl.ds(0, 16)]
                    ev, od = plsc.unpack(
                        plsc.bitcast(w, jnp.bfloat16),
                        format=plsc.PackFormat.INTERLEAVED)
                    q = u % 4
                    accs[2 * q] = accs[2 * q] + ev
                    accs[2 * q + 1] = accs[2 * q + 1] + od
                return tuple(accs)

            zero = jnp.zeros((16,), jnp.float32)
            accs = lax.fori_loop(0, HIST // UNROLL, row_body, (zero,) * 8)
            sum_v[s, pl.ds(0, 16)] = (accs[0] + accs[2]) + (accs[4] + accs[6])
            sum_v[s, pl.ds(16, 16)] = (accs[1] + accs[3]) + (accs[5] + accs[7])
        off = base + c * CH
        pltpu.sync_copy(sum_v, out_hbm.at[pl.ds(off, CH)])

    pltpu.sync_copy(xf_hbm.at[pl.ds(base, CH), :], idxf_v.at[0])
    convert(0)
    for _isl, _rsl in gather_list(0):
        pltpu.async_copy(emb_hbm.at[_isl], _rsl, sems[0])
    stage(1, 1)

    def pair_body(c2, carry):
        for b in range(2):
            c = c2 + b
            fire(c + 1, 1 - b)
            stage(c + 2, b)
            drain(b)
            consume(c, b)
        return carry

    lax.fori_loop(0, NCHUNK // 2, lambda i, cr: pair_body(i * 2, cr), 0)


_TC_BLK = 2048


def _tc_head_body(x_ref, sum_ref, w_ref, b_ref, o_ref):
    cnt = jnp.sum((x_ref[...] != 0).astype(jnp.float32), axis=1,
                  keepdims=True)
    mean = sum_ref[...] * (1.0 / jnp.maximum(cnt, 1.0))
    o_ref[...] = (
        jnp.dot(mean, w_ref[...], preferred_element_type=jnp.float32)
        + b_ref[...])


_tc_head = pl.pallas_call(
    _tc_head_body,
    grid=(B // _TC_BLK,),
    in_specs=[
        pl.BlockSpec((_TC_BLK, HIST), lambda i: (i, 0)),
        pl.BlockSpec((_TC_BLK, D), lambda i: (i, 0)),
        pl.BlockSpec((D, CLS), lambda i: (0, 0)),
        pl.BlockSpec((1, CLS), lambda i: (0, 0)),
    ],
    out_specs=pl.BlockSpec((_TC_BLK, CLS), lambda i: (i, 0)),
    out_shape=jax.ShapeDtypeStruct((B, CLS), jnp.float32),
)


def kernel(x, emb, W, b):
    x = x.astype(jnp.int32)
    xf = jax.lax.bitcast_convert_type(x, jnp.float32)
    emb_lin = _sc_relayout(emb.T, emb[NFULL * TBLK:, :]).reshape(VOCAB,
                                                                 D // 2)
    summed = _sc_bag_sum(xf, emb_lin)
    w_eo = jnp.concatenate([W[0::2, :], W[1::2, :]], axis=0)
    return _tc_head(x, summed, w_eo, b.reshape(1, CLS))
